# baseline (speedup 1.0000x reference)
import jax
import jax.numpy as jnp
from jax.experimental import pallas as pl


def kernel(item_seqs, emb):
    batch, seq_len = item_seqs.shape
    hidden = emb.shape[1]
    bb = 16

    def body(emb_ref, out_ref):
        out_ref[...] = jnp.broadcast_to(
            emb_ref[...][None], (bb, seq_len, hidden)
        )

    out = pl.pallas_call(
        body,
        grid=(batch // bb,),
        in_specs=[pl.BlockSpec((seq_len, hidden), lambda i: (0, 0))],
        out_specs=pl.BlockSpec((bb, seq_len, hidden), lambda i: (i, 0, 0)),
        out_shape=jax.ShapeDtypeStruct((batch, seq_len, hidden), jnp.float32),
    )(emb[:seq_len])
    return out


# speedup vs baseline: 1.1136x; 1.1136x over previous
import functools

import jax
import jax.numpy as jnp
from jax import lax
from jax.experimental import pallas as pl
from jax.experimental.pallas import tpu as pltpu
from jax.experimental.pallas import tpu_sc as plsc


def _gather_positions_sc(emb, seq_len):
    vocab, hidden = emb.shape
    lanes = 16
    n_pad = (seq_len + lanes - 1) // lanes * lanes
    mesh = plsc.VectorSubcoreMesh(core_axis_name="c", subcore_axis_name="s")

    @functools.partial(
        pl.kernel,
        mesh=mesh,
        out_type=jax.ShapeDtypeStruct((seq_len, hidden), jnp.float32),
        scratch_types=[
            pltpu.VMEM((n_pad,), jnp.int32),
            pltpu.VMEM((n_pad, hidden), jnp.float32),
            pltpu.SemaphoreType.DMA,
        ],
    )
    def gather(emb_hbm, out_hbm, idx_v, rows_v, sem):
        wid = lax.axis_index("s") * 2 + lax.axis_index("c")

        @pl.when(wid == 0)
        def _():
            for c in range(n_pad // lanes):
                pos = lax.iota(jnp.int32, lanes) + c * lanes
                idx_v[pl.ds(c * lanes, lanes)] = jnp.minimum(pos, vocab - 1)
            pltpu.async_copy(emb_hbm.at[idx_v], rows_v, sem).wait()
            pltpu.sync_copy(rows_v.at[pl.ds(0, seq_len)], out_hbm)

    return gather(emb)


def kernel(item_seqs, emb):
    batch, seq_len = item_seqs.shape
    hidden = emb.shape[1]

    pos_emb = _gather_positions_sc(emb, seq_len)

    bb = 64

    def body(pos_ref, out_ref):
        out_ref[...] = jnp.broadcast_to(
            pos_ref[...][None], (bb, seq_len, hidden)
        )

    out = pl.pallas_call(
        body,
        grid=(batch // bb,),
        in_specs=[pl.BlockSpec((seq_len, hidden), lambda i: (0, 0))],
        out_specs=pl.BlockSpec((bb, seq_len, hidden), lambda i: (i, 0, 0)),
        out_shape=jax.ShapeDtypeStruct((batch, seq_len, hidden), jnp.float32),
    )(pos_emb)
    return out


# speedup vs baseline: 1.1246x; 1.0099x over previous
import functools

import jax
import jax.numpy as jnp
from jax import lax
from jax.experimental import pallas as pl
from jax.experimental.pallas import tpu as pltpu
from jax.experimental.pallas import tpu_sc as plsc


def _gather_positions_sc(emb, seq_len):
    vocab, hidden = emb.shape
    lanes = 16
    n_pad = (seq_len + lanes - 1) // lanes * lanes
    mesh = plsc.VectorSubcoreMesh(
        core_axis_name="c", subcore_axis_name="s", num_cores=1
    )

    @functools.partial(
        pl.kernel,
        mesh=mesh,
        out_type=jax.ShapeDtypeStruct((seq_len, hidden), jnp.float32),
        scratch_types=[
            pltpu.VMEM((n_pad,), jnp.int32),
            pltpu.VMEM((n_pad, hidden), jnp.float32),
            pltpu.SemaphoreType.DMA,
        ],
    )
    def gather(emb_hbm, out_hbm, idx_v, rows_v, sem):
        wid = lax.axis_index("s") * 2 + lax.axis_index("c")

        @pl.when(wid == 0)
        def _():
            for c in range(n_pad // lanes):
                pos = lax.iota(jnp.int32, lanes) + c * lanes
                idx_v[pl.ds(c * lanes, lanes)] = jnp.minimum(pos, vocab - 1)
            pltpu.async_copy(emb_hbm.at[idx_v], rows_v, sem).wait()
            pltpu.sync_copy(rows_v.at[pl.ds(0, seq_len)], out_hbm)

    return gather(emb)


def kernel(item_seqs, emb):
    batch, seq_len = item_seqs.shape
    hidden = emb.shape[1]

    pos_emb = _gather_positions_sc(emb, seq_len)

    bb = 64

    def body(pos_ref, out_ref):
        out_ref[...] = jnp.broadcast_to(
            pos_ref[...][None], (bb, seq_len, hidden)
        )

    out = pl.pallas_call(
        body,
        grid=(batch // bb,),
        in_specs=[pl.BlockSpec((seq_len, hidden), lambda i: (0, 0))],
        out_specs=pl.BlockSpec((bb, seq_len, hidden), lambda i: (i, 0, 0)),
        out_shape=jax.ShapeDtypeStruct((batch, seq_len, hidden), jnp.float32),
    )(pos_emb)
    return out


# speedup vs baseline: 1.1596x; 1.0311x over previous
import functools

import jax
import jax.numpy as jnp
from jax import lax
from jax.experimental import pallas as pl
from jax.experimental.pallas import tpu as pltpu
from jax.experimental.pallas import tpu_sc as plsc


def _gather_positions_sc(emb, seq_len):
    vocab, hidden = emb.shape
    lanes = 16
    n_pad = (seq_len + lanes - 1) // lanes * lanes
    mesh = plsc.VectorSubcoreMesh(
        core_axis_name="c", subcore_axis_name="s", num_cores=1
    )

    @functools.partial(
        pl.kernel,
        mesh=mesh,
        out_type=jax.ShapeDtypeStruct((seq_len, hidden), jnp.float32),
        scratch_types=[
            pltpu.VMEM((n_pad,), jnp.int32),
            pltpu.VMEM((n_pad, hidden), jnp.float32),
            pltpu.SemaphoreType.DMA,
        ],
    )
    def gather(emb_hbm, out_hbm, idx_v, rows_v, sem):
        wid = lax.axis_index("s") + lax.axis_index("c")

        @pl.when(wid == 0)
        def _():
            for c in range(n_pad // lanes):
                pos = lax.iota(jnp.int32, lanes) + c * lanes
                idx_v[pl.ds(c * lanes, lanes)] = jnp.minimum(pos, vocab - 1)
            pltpu.async_copy(emb_hbm.at[idx_v], rows_v, sem).wait()
            pltpu.sync_copy(rows_v.at[pl.ds(0, seq_len)], out_hbm)

    return gather(emb)


def kernel(item_seqs, emb):
    batch, seq_len = item_seqs.shape
    hidden = emb.shape[1]
    bb = 64
    tail_blocks = 1
    main = batch - tail_blocks * bb
    out_shape = jax.ShapeDtypeStruct((batch, seq_len, hidden), jnp.float32)

    pos_emb = _gather_positions_sc(emb, seq_len)

    def body(src_ref, out_ref):
        out_ref[...] = jnp.broadcast_to(
            src_ref[...][None], (bb, seq_len, hidden)
        )

    part_a = pl.pallas_call(
        body,
        grid=(main // bb,),
        in_specs=[pl.BlockSpec((seq_len, hidden), lambda i: (0, 0))],
        out_specs=pl.BlockSpec((bb, seq_len, hidden), lambda i: (i, 0, 0)),
        out_shape=out_shape,
    )(emb[:seq_len])

    def body_tail(src_ref, alias_ref, out_ref):
        del alias_ref
        out_ref[...] = jnp.broadcast_to(
            src_ref[...][None], (bb, seq_len, hidden)
        )

    out = pl.pallas_call(
        body_tail,
        grid=(tail_blocks,),
        in_specs=[
            pl.BlockSpec((seq_len, hidden), lambda i: (0, 0)),
            pl.BlockSpec(memory_space=pl.ANY),
        ],
        out_specs=pl.BlockSpec(
            (bb, seq_len, hidden), lambda i: (main // bb + i, 0, 0)
        ),
        out_shape=out_shape,
        input_output_aliases={1: 0},
    )(pos_emb, part_a)
    return out


# speedup vs baseline: 1.2855x; 1.1086x over previous
import functools

import jax
import jax.numpy as jnp
from jax import lax
from jax.experimental import pallas as pl
from jax.experimental.pallas import tpu as pltpu
from jax.experimental.pallas import tpu_sc as plsc


def _gather_positions_sc(emb, seq_len):
    vocab, hidden = emb.shape
    lanes = 16
    n_pad = (seq_len + lanes - 1) // lanes * lanes
    mesh = plsc.VectorSubcoreMesh(
        core_axis_name="c", subcore_axis_name="s", num_cores=1
    )

    @functools.partial(
        pl.kernel,
        mesh=mesh,
        out_type=jax.ShapeDtypeStruct((seq_len, hidden), jnp.float32),
        scratch_types=[
            pltpu.VMEM((n_pad,), jnp.int32),
            pltpu.VMEM((n_pad, hidden), jnp.float32),
            pltpu.SemaphoreType.DMA,
        ],
    )
    def gather(emb_hbm, out_hbm, idx_v, rows_v, sem):
        wid = lax.axis_index("s") + lax.axis_index("c")

        @pl.when(wid == 0)
        def _():
            for c in range(n_pad // lanes):
                pos = lax.iota(jnp.int32, lanes) + c * lanes
                idx_v[pl.ds(c * lanes, lanes)] = jnp.minimum(pos, vocab - 1)
            pltpu.async_copy(emb_hbm.at[idx_v], rows_v, sem).wait()
            pltpu.sync_copy(rows_v.at[pl.ds(0, seq_len)], out_hbm)

    return gather(emb)


def kernel(item_seqs, emb):
    batch, seq_len = item_seqs.shape
    hidden = emb.shape[1]
    bb = 64
    tail_blocks = 1
    main = batch - tail_blocks * bb
    out_shape = jax.ShapeDtypeStruct((batch, seq_len, hidden), jnp.float32)

    pos_emb = emb[:seq_len]

    def body(src_ref, out_ref):
        out_ref[...] = jnp.broadcast_to(
            src_ref[...][None], (bb, seq_len, hidden)
        )

    part_a = pl.pallas_call(
        body,
        grid=(main // bb,),
        in_specs=[pl.BlockSpec((seq_len, hidden), lambda i: (0, 0))],
        out_specs=pl.BlockSpec((bb, seq_len, hidden), lambda i: (i, 0, 0)),
        out_shape=out_shape,
    )(emb[:seq_len])

    def body_tail(src_ref, alias_ref, out_ref):
        del alias_ref
        out_ref[...] = jnp.broadcast_to(
            src_ref[...][None], (bb, seq_len, hidden)
        )

    out = pl.pallas_call(
        body_tail,
        grid=(tail_blocks,),
        in_specs=[
            pl.BlockSpec((seq_len, hidden), lambda i: (0, 0)),
            pl.BlockSpec(memory_space=pl.ANY),
        ],
        out_specs=pl.BlockSpec(
            (bb, seq_len, hidden), lambda i: (main // bb + i, 0, 0)
        ),
        out_shape=out_shape,
        input_output_aliases={1: 0},
    )(pos_emb, part_a)
    return out
